# baseline (speedup 1.0000x reference)
import functools

import jax
import jax.numpy as jnp
from jax import lax
from jax.experimental import pallas as pl
from jax.experimental.pallas import tpu as pltpu
from jax.experimental.pallas import tpu_sc as plsc

F = 256
K = 8192
N = 4096

BN = 128
BK = 2048
IT = N // BN
JT = K // BK

NC = 2
NS = 16
NW = NC * NS
BPW = N // NW


def _argmin_onehot_body(z_ref, e_ref, idx_ref, enc_ref, c_ref, cols_ref):
    i = pl.program_id(0)

    @pl.when(i == 0)
    def _():
        e = e_ref[...]
        c_ref[...] = jnp.sum(e * e, axis=0, keepdims=True)
        cols_ref[...] = lax.broadcasted_iota(
            jnp.int32, (1, K), 1).astype(jnp.float32)

    z = z_ref[...]
    a = jnp.sum(z * z, axis=1, keepdims=True)
    b2 = jnp.dot(z * (-2.0), e_ref[...], preferred_element_type=jnp.float32)
    d = (a + b2) + c_ref[...]
    m = jnp.min(d, axis=1, keepdims=True)
    cols = cols_ref[...]
    loc = jnp.min(jnp.where(d == m, cols, float(K)), axis=1, keepdims=True)
    idx_ref[...] = loc.astype(jnp.int32)
    enc_ref[...] = (cols == loc).astype(jnp.float32)


NCH = 4
CH = BPW // NCH


def _sc_body(et_ref, idx_ref, ze_ref, zq_ref, diff_ref,
             idx_v, rows_v, ze_v, g0, g1, g2, g3, zsem, osem):
    wid = lax.axis_index("s") * NC + lax.axis_index("c")
    base = wid * BPW
    pltpu.sync_copy(idx_ref.at[pl.ds(base, BPW)], idx_v)
    zc = pltpu.async_copy(ze_ref.at[pl.ds(base, BPW)], ze_v, zsem)
    gsems = (g0, g1, g2, g3)
    gathers = [
        pltpu.async_copy(
            et_ref.at[idx_v.at[pl.ds(k * CH, CH)]],
            rows_v.at[pl.ds(k * CH, CH)], gsems[k])
        for k in range(NCH)
    ]
    zc.wait()

    def row(r, carry):
        for c0 in range(F // 16):
            sl = pl.ds(c0 * 16, 16)
            zq = rows_v[r, sl]
            ze = ze_v[r, sl]
            t = zq - ze
            rows_v[r, sl] = ze + t
            ze_v[r, sl] = t * t
        return carry

    stores = []
    for k in range(NCH):
        gathers[k].wait()
        lax.fori_loop(k * CH, (k + 1) * CH, row, 0)
        sl = pl.ds(k * CH, CH)
        osl = pl.ds(base + k * CH, CH)
        stores.append(pltpu.async_copy(rows_v.at[sl], zq_ref.at[osl], osem))
        stores.append(pltpu.async_copy(ze_v.at[sl], diff_ref.at[osl], osem))
    for s in stores:
        s.wait()


def _sc_quantise(e_t, idx_flat, z_e):
    run = functools.partial(
        pl.kernel,
        out_type=(jax.ShapeDtypeStruct((N, F), jnp.float32),
                  jax.ShapeDtypeStruct((N, F), jnp.float32)),
        mesh=plsc.VectorSubcoreMesh(core_axis_name="c", subcore_axis_name="s"),
        scratch_types=[
            pltpu.VMEM((BPW,), jnp.int32),
            pltpu.VMEM((BPW, F), jnp.float32),
            pltpu.VMEM((BPW, F), jnp.float32),
            pltpu.SemaphoreType.DMA,
            pltpu.SemaphoreType.DMA,
            pltpu.SemaphoreType.DMA,
            pltpu.SemaphoreType.DMA,
            pltpu.SemaphoreType.DMA,
            pltpu.SemaphoreType.DMA,
        ],
    )(_sc_body)
    return run(e_t, idx_flat, z_e)


def kernel(z_e, embedding_weight):
    idx, enc = pl.pallas_call(
        _argmin_onehot_body,
        grid=(IT,),
        in_specs=[
            pl.BlockSpec((BN, F), lambda i: (i, 0)),
            pl.BlockSpec((F, K), lambda i: (0, 0)),
        ],
        out_specs=[
            pl.BlockSpec((BN, 1), lambda i: (i, 0)),
            pl.BlockSpec((BN, K), lambda i: (i, 0)),
        ],
        out_shape=[
            jax.ShapeDtypeStruct((N, 1), jnp.int32),
            jax.ShapeDtypeStruct((N, K), jnp.float32),
        ],
        scratch_shapes=[
            pltpu.VMEM((1, K), jnp.float32),
            pltpu.VMEM((1, K), jnp.float32),
        ],
    )(z_e, embedding_weight)

    e_t = embedding_weight.T
    zq_st, diff = _sc_quantise(e_t, idx[:, 0], z_e)
    return (zq_st, idx, enc, diff)


# speedup vs baseline: 1.1119x; 1.1119x over previous
import functools

import jax
import jax.numpy as jnp
from jax import lax
from jax.experimental import pallas as pl
from jax.experimental.pallas import tpu as pltpu
from jax.experimental.pallas import tpu_sc as plsc

F = 256
K = 8192
N = 4096

BN = 256
BK = 2048
IT = N // BN
JT = K // BK

NC = 2
NS = 16
NW = NC * NS
BPW = N // NW


def _argmin_onehot_body(z_ref, e_ref, idx_ref, enc_ref, c_ref, cols_ref):
    i = pl.program_id(0)

    @pl.when(i == 0)
    def _():
        e = e_ref[...]
        c_ref[...] = jnp.sum(e * e, axis=0, keepdims=True)
        cols_ref[...] = lax.broadcasted_iota(
            jnp.int32, (1, K), 1).astype(jnp.float32)

    z = z_ref[...]
    a = jnp.sum(z * z, axis=1, keepdims=True)
    b2 = jnp.dot(z * (-2.0), e_ref[...], preferred_element_type=jnp.float32)
    d = (a + b2) + c_ref[...]
    m = jnp.min(d, axis=1, keepdims=True)
    cols_i = lax.broadcasted_iota(jnp.int32, d.shape, 1)
    cols = cols_i.astype(jnp.float32)
    loc = jnp.min(jnp.where(d == m, cols, float(K)), axis=1, keepdims=True)
    idx_ref[...] = loc.astype(jnp.int32)
    enc_ref[...] = (cols == loc).astype(jnp.float32)


NCH = 4
CH = BPW // NCH


def _sc_body(et_ref, idx_ref, ze_ref, zq_ref, diff_ref,
             idx_v, rows_v, ze_v, g0, g1, g2, g3, zsem, osem):
    wid = lax.axis_index("s") * NC + lax.axis_index("c")
    base = wid * BPW
    pltpu.sync_copy(idx_ref.at[pl.ds(base, BPW)], idx_v)
    zc = pltpu.async_copy(ze_ref.at[pl.ds(base, BPW)], ze_v, zsem)
    gsems = (g0, g1, g2, g3)
    gathers = [
        pltpu.async_copy(
            et_ref.at[idx_v.at[pl.ds(k * CH, CH)]],
            rows_v.at[pl.ds(k * CH, CH)], gsems[k])
        for k in range(NCH)
    ]
    zc.wait()

    def row(r, carry):
        for c0 in range(F // 16):
            sl = pl.ds(c0 * 16, 16)
            zq = rows_v[r, sl]
            ze = ze_v[r, sl]
            t = zq - ze
            rows_v[r, sl] = ze + t
            ze_v[r, sl] = t * t
        return carry

    stores = []
    for k in range(NCH):
        gathers[k].wait()
        lax.fori_loop(k * CH, (k + 1) * CH, row, 0)
        sl = pl.ds(k * CH, CH)
        osl = pl.ds(base + k * CH, CH)
        stores.append(pltpu.async_copy(rows_v.at[sl], zq_ref.at[osl], osem))
        stores.append(pltpu.async_copy(ze_v.at[sl], diff_ref.at[osl], osem))
    for s in stores:
        s.wait()


def _sc_quantise(e_t, idx_flat, z_e):
    run = functools.partial(
        pl.kernel,
        out_type=(jax.ShapeDtypeStruct((N, F), jnp.float32),
                  jax.ShapeDtypeStruct((N, F), jnp.float32)),
        mesh=plsc.VectorSubcoreMesh(core_axis_name="c", subcore_axis_name="s"),
        scratch_types=[
            pltpu.VMEM((BPW,), jnp.int32),
            pltpu.VMEM((BPW, F), jnp.float32),
            pltpu.VMEM((BPW, F), jnp.float32),
            pltpu.SemaphoreType.DMA,
            pltpu.SemaphoreType.DMA,
            pltpu.SemaphoreType.DMA,
            pltpu.SemaphoreType.DMA,
            pltpu.SemaphoreType.DMA,
            pltpu.SemaphoreType.DMA,
        ],
    )(_sc_body)
    return run(e_t, idx_flat, z_e)


def kernel(z_e, embedding_weight):
    idx, enc = pl.pallas_call(
        _argmin_onehot_body,
        grid=(IT,),
        in_specs=[
            pl.BlockSpec((BN, F), lambda i: (i, 0)),
            pl.BlockSpec((F, K), lambda i: (0, 0)),
        ],
        out_specs=[
            pl.BlockSpec((BN, 1), lambda i: (i, 0)),
            pl.BlockSpec((BN, K), lambda i: (i, 0)),
        ],
        out_shape=[
            jax.ShapeDtypeStruct((N, 1), jnp.int32),
            jax.ShapeDtypeStruct((N, K), jnp.float32),
        ],
        scratch_shapes=[
            pltpu.VMEM((1, K), jnp.float32),
            pltpu.VMEM((1, K), jnp.float32),
        ],
    )(z_e, embedding_weight)

    e_t = embedding_weight.T
    zq_st, diff = _sc_quantise(e_t, idx[:, 0], z_e)
    return (zq_st, idx, enc, diff)


# speedup vs baseline: 1.1155x; 1.0033x over previous
import functools

import jax
import jax.numpy as jnp
from jax import lax
from jax.experimental import pallas as pl
from jax.experimental.pallas import tpu as pltpu
from jax.experimental.pallas import tpu_sc as plsc

F = 256
K = 8192
N = 4096

BN = 256
BK = 2048
IT = N // BN
JT = K // BK

NC = 2
NS = 16
NW = NC * NS
BPW = N // NW


def _argmin_onehot_body(z_ref, e_ref, idx_ref, enc_ref, c_ref):
    i = pl.program_id(0)

    @pl.when(i == 0)
    def _():
        e = e_ref[...]
        c_ref[...] = jnp.sum(e * e, axis=0, keepdims=True)

    z = z_ref[...]
    a = jnp.sum(z * z, axis=1, keepdims=True)
    b2 = jnp.dot(z * (-2.0), e_ref[...], preferred_element_type=jnp.float32)
    d = (a + b2) + c_ref[...]
    m = jnp.min(d, axis=1, keepdims=True)
    cols_i = lax.broadcasted_iota(jnp.int32, d.shape, 1)
    cols = cols_i.astype(jnp.float32)
    loc = jnp.min(jnp.where(d == m, cols, float(K)), axis=1, keepdims=True)
    idx_ref[...] = loc.astype(jnp.int32)
    enc_ref[...] = (cols == loc).astype(jnp.float32)


NCH = 4
CH = BPW // NCH


def _sc_body(et_ref, idx_ref, ze_ref, zq_ref, diff_ref,
             idx_v, rows_v, ze_v, g0, g1, g2, g3, zsem, osem):
    wid = lax.axis_index("s") * NC + lax.axis_index("c")
    base = wid * BPW
    pltpu.sync_copy(idx_ref.at[pl.ds(base, BPW)], idx_v)
    zc = pltpu.async_copy(ze_ref.at[pl.ds(base, BPW)], ze_v, zsem)
    gsems = (g0, g1, g2, g3)
    gathers = [
        pltpu.async_copy(
            et_ref.at[idx_v.at[pl.ds(k * CH, CH)]],
            rows_v.at[pl.ds(k * CH, CH)], gsems[k])
        for k in range(NCH)
    ]
    zc.wait()

    def row(r, carry):
        for c0 in range(F // 16):
            sl = pl.ds(c0 * 16, 16)
            zq = rows_v[r, sl]
            ze = ze_v[r, sl]
            t = zq - ze
            rows_v[r, sl] = ze + t
            ze_v[r, sl] = t * t
        return carry

    stores = []
    for k in range(NCH):
        gathers[k].wait()
        lax.fori_loop(k * CH, (k + 1) * CH, row, 0)
        sl = pl.ds(k * CH, CH)
        osl = pl.ds(base + k * CH, CH)
        stores.append(pltpu.async_copy(rows_v.at[sl], zq_ref.at[osl], osem))
        stores.append(pltpu.async_copy(ze_v.at[sl], diff_ref.at[osl], osem))
    for s in stores:
        s.wait()


def _sc_quantise(e_t, idx_flat, z_e):
    run = functools.partial(
        pl.kernel,
        out_type=(jax.ShapeDtypeStruct((N, F), jnp.float32),
                  jax.ShapeDtypeStruct((N, F), jnp.float32)),
        mesh=plsc.VectorSubcoreMesh(core_axis_name="c", subcore_axis_name="s"),
        scratch_types=[
            pltpu.VMEM((BPW,), jnp.int32),
            pltpu.VMEM((BPW, F), jnp.float32),
            pltpu.VMEM((BPW, F), jnp.float32),
            pltpu.SemaphoreType.DMA,
            pltpu.SemaphoreType.DMA,
            pltpu.SemaphoreType.DMA,
            pltpu.SemaphoreType.DMA,
            pltpu.SemaphoreType.DMA,
            pltpu.SemaphoreType.DMA,
        ],
    )(_sc_body)
    return run(e_t, idx_flat, z_e)


def kernel(z_e, embedding_weight):
    idx, enc = pl.pallas_call(
        _argmin_onehot_body,
        grid=(IT,),
        in_specs=[
            pl.BlockSpec((BN, F), lambda i: (i, 0)),
            pl.BlockSpec((F, K), lambda i: (0, 0)),
        ],
        out_specs=[
            pl.BlockSpec((BN, 1), lambda i: (i, 0)),
            pl.BlockSpec((BN, K), lambda i: (i, 0)),
        ],
        out_shape=[
            jax.ShapeDtypeStruct((N, 1), jnp.int32),
            jax.ShapeDtypeStruct((N, K), jnp.float32),
        ],
        scratch_shapes=[pltpu.VMEM((1, K), jnp.float32)],
    )(z_e, embedding_weight)

    e_t = embedding_weight.T
    zq_st, diff = _sc_quantise(e_t, idx[:, 0], z_e)
    return (zq_st, idx, enc, diff)
